# baseline (speedup 1.0000x reference)
import functools

import jax
import jax.numpy as jnp
from jax import lax
from jax.experimental import pallas as pl
from jax.experimental.pallas import tpu as pltpu
from jax.experimental.pallas import tpu_sc as plsc

_INFO = plsc.get_sparse_core_info()
_NC, _NS, _L = _INFO.num_cores, _INFO.num_subcores, _INFO.num_lanes
_NW = _NC * _NS


@functools.lru_cache(maxsize=None)
def _make_sc_gather_sum(n_total: int, vocab: int):
    per_w = n_total // _NW
    n_vec = per_w // _L
    mesh = plsc.VectorSubcoreMesh(core_axis_name="c", subcore_axis_name="s")

    @functools.partial(
        pl.kernel,
        mesh=mesh,
        out_type=jax.ShapeDtypeStruct((_NW, _L), jnp.float32),
        scratch_types=[
            pltpu.VMEM((per_w,), jnp.int32),
            pltpu.VMEM((per_w,), jnp.int32),
            pltpu.VMEM((per_w,), jnp.float32),
            pltpu.VMEM((_L,), jnp.float32),
            pltpu.SemaphoreType.DMA,
        ],
    )
    def sc_kernel(logits_hbm, tid_hbm, out_hbm,
                  tid_v, idx_v, vals_v, stage_v, sem):
        cid = lax.axis_index("c")
        sid = lax.axis_index("s")
        wid = sid * _NC + cid
        base = wid * per_w

        pltpu.sync_copy(tid_hbm.at[pl.ds(base, per_w)], tid_v)

        lane = lax.iota(jnp.int32, _L)
        for j in range(n_vec):
            row0 = (base + j * _L) * vocab
            idx_v[pl.ds(j * _L, _L)] = (
                row0 + lane * vocab + tid_v[pl.ds(j * _L, _L)]
            )

        pltpu.async_copy(logits_hbm.at[idx_v], vals_v, sem).wait()

        acc = jnp.zeros((_L,), jnp.float32)
        for j in range(n_vec):
            acc = acc + vals_v[pl.ds(j * _L, _L)]
        stage_v[...] = acc

        pltpu.sync_copy(stage_v, out_hbm.at[wid])

    return sc_kernel


def kernel(logits, target_ids):
    batch, seq, vocab = logits.shape
    flat_logits = logits.reshape((-1,))
    flat_tid = target_ids.reshape((-1,)).astype(jnp.int32)
    partials = _make_sc_gather_sum(batch * seq, vocab)(flat_logits, flat_tid)
    return -(jnp.sum(partials) / batch)


# speedup vs baseline: 10.5440x; 10.5440x over previous
import functools

import jax
import jax.numpy as jnp
from jax import lax
from jax.experimental import pallas as pl
from jax.experimental.pallas import tpu as pltpu
from jax.experimental.pallas import tpu_sc as plsc

_INFO = plsc.get_sparse_core_info()
_NC, _NS, _L = _INFO.num_cores, _INFO.num_subcores, _INFO.num_lanes
_NW = _NC * _NS


@functools.lru_cache(maxsize=None)
def _make_sc_gather_sum(n_rows: int, vocab: int):
    per_w = n_rows // _NW
    n_vec = per_w // _L
    mesh = plsc.VectorSubcoreMesh(core_axis_name="c", subcore_axis_name="s")

    @functools.partial(
        pl.kernel,
        mesh=mesh,
        compiler_params=pltpu.CompilerParams(needs_layout_passes=False),
        out_type=jax.ShapeDtypeStruct((_NW, _L), jnp.float32),
        scratch_types=[
            pltpu.VMEM((per_w,), jnp.int32),
            pltpu.VMEM((_L * 8, 128), jnp.float32),
            pltpu.VMEM((_L,), jnp.float32),
            pltpu.SemaphoreType.DMA,
        ],
    )
    def sc_kernel(logits_hbm, tid_hbm, out_hbm,
                  tid_v, vals_v, stage_v, sem):
        cid = lax.axis_index("c")
        sid = lax.axis_index("s")
        wid = sid * _NC + cid
        base = wid * per_w

        pltpu.sync_copy(tid_hbm.at[pl.ds(base, per_w)], tid_v)

        lane = lax.iota(jnp.int32, _L)
        rows = lane * 8 + (lane & 7)
        acc = jnp.zeros((_L,), jnp.float32)
        for g in range(n_vec):
            vec = tid_v[pl.ds(g * _L, _L)]
            handles = []
            for k in range(_L):
                j = g * _L + k
                v = lax.reshape(lax.slice(vec, (k,), (k + 1,)), ())
                c0 = pl.multiple_of(jnp.bitwise_and(v, jnp.int32(-128)), 128)
                t0 = pl.multiple_of((base + j) & jnp.int32(-8), 8)
                handles.append(pltpu.async_copy(
                    logits_hbm.at[pl.ds(t0, 8), pl.ds(c0, 128)],
                    vals_v.at[pl.ds(k * 8, 8)],
                    sem,
                ))
            for h in handles:
                h.wait()
            acc = acc + plsc.load_gather(vals_v, [rows, vec & 127])
        stage_v[...] = acc

        pltpu.sync_copy(stage_v, out_hbm.at[wid])

    return sc_kernel


def kernel(logits, target_ids):
    batch, seq, vocab = logits.shape
    logits2d = logits.reshape((batch * seq, vocab))
    flat_tid = target_ids.reshape((-1,)).astype(jnp.int32)
    partials = _make_sc_gather_sum(batch * seq, vocab)(logits2d, flat_tid)
    return -(jnp.sum(partials) / batch)


# speedup vs baseline: 11.0874x; 1.0515x over previous
import functools

import jax
import jax.numpy as jnp
from jax import lax
from jax.experimental import pallas as pl
from jax.experimental.pallas import tpu as pltpu
from jax.experimental.pallas import tpu_sc as plsc

_INFO = plsc.get_sparse_core_info()
_NC, _NS, _L = _INFO.num_cores, _INFO.num_subcores, _INFO.num_lanes
_NW = _NC * _NS


@functools.lru_cache(maxsize=None)
def _make_sc_gather_sum(batch: int, seq: int, vocab: int):
    n_rows = batch * seq
    per_w = n_rows // _NW
    n_vec = per_w // _L
    mesh = plsc.VectorSubcoreMesh(core_axis_name="c", subcore_axis_name="s")

    @functools.partial(
        pl.kernel,
        mesh=mesh,
        compiler_params=pltpu.CompilerParams(needs_layout_passes=False),
        out_type=jax.ShapeDtypeStruct((_NW, _L), jnp.float32),
        scratch_types=[
            pltpu.VMEM((batch, seq), jnp.int32),
            pltpu.VMEM((_L * 8, 128), jnp.float32),
            pltpu.VMEM((_L * 8, 128), jnp.float32),
            pltpu.VMEM((_L,), jnp.float32),
            pltpu.SemaphoreType.DMA,
            pltpu.SemaphoreType.DMA,
        ],
    )
    def sc_kernel(logits_hbm, tid_hbm, out_hbm,
                  tid_v, buf_a, buf_b, stage_v, sem_a, sem_b):
        cid = lax.axis_index("c")
        sid = lax.axis_index("s")
        wid = sid * _NC + cid
        base = wid * per_w
        b_idx = base // seq

        pltpu.sync_copy(tid_hbm, tid_v)

        bufs = (buf_a, buf_b)
        sems = (sem_a, sem_b)

        def fire(g):
            s0 = base % seq + g * _L
            vec = tid_v[b_idx, pl.ds(s0, _L)]
            buf, sem = bufs[g % 2], sems[g % 2]
            handles = []
            for k in range(_L):
                j = g * _L + k
                v = lax.reshape(lax.slice(vec, (k,), (k + 1,)), ())
                c0 = pl.multiple_of(jnp.bitwise_and(v, jnp.int32(-128)), 128)
                t0 = pl.multiple_of((base + j) & jnp.int32(-8), 8)
                handles.append(pltpu.async_copy(
                    logits_hbm.at[pl.ds(t0, 8), pl.ds(c0, 128)],
                    buf.at[pl.ds(k * 8, 8)],
                    sem,
                ))
            return vec, handles

        lane = lax.iota(jnp.int32, _L)
        rows = lane * 8 + (lane & 7)
        acc = jnp.zeros((_L,), jnp.float32)
        pending = fire(0)
        for g in range(n_vec):
            vec, handles = pending
            if g + 1 < n_vec:
                pending = fire(g + 1)
            for h in handles:
                h.wait()
            acc = acc + plsc.load_gather(bufs[g % 2], [rows, vec & 127])
        stage_v[...] = acc

        pltpu.sync_copy(stage_v, out_hbm.at[wid])

    return sc_kernel


def kernel(logits, target_ids):
    batch, seq, vocab = logits.shape
    logits2d = logits.reshape((batch * seq, vocab))
    tid = target_ids.astype(jnp.int32)
    partials = _make_sc_gather_sum(batch, seq, vocab)(logits2d, tid)
    return -(jnp.sum(partials) / batch)


# speedup vs baseline: 11.2830x; 1.0176x over previous
import functools

import jax
import jax.numpy as jnp
from jax import lax
from jax.experimental import pallas as pl
from jax.experimental.pallas import tpu as pltpu
from jax.experimental.pallas import tpu_sc as plsc

_INFO = plsc.get_sparse_core_info()
_NC, _NS, _L = _INFO.num_cores, _INFO.num_subcores, _INFO.num_lanes
_NW = _NC * _NS


@functools.lru_cache(maxsize=None)
def _make_sc_gather_sum(batch: int, seq: int, vocab: int):
    n_rows = batch * seq
    per_w = n_rows // _NW
    n_vec = per_w // _L
    mesh = plsc.VectorSubcoreMesh(core_axis_name="c", subcore_axis_name="s")

    @functools.partial(
        pl.kernel,
        mesh=mesh,
        compiler_params=pltpu.CompilerParams(needs_layout_passes=False),
        out_type=jax.ShapeDtypeStruct((_NW, _L), jnp.float32),
        scratch_types=[
            pltpu.VMEM((batch, seq), jnp.int32),
            pltpu.VMEM((_L * 8, 128), jnp.float32),
            pltpu.VMEM((_L * 8, 128), jnp.float32),
            pltpu.VMEM((_L * 8, 128), jnp.float32),
            pltpu.VMEM((_L * 8, 128), jnp.float32),
            pltpu.VMEM((_L,), jnp.float32),
            pltpu.SemaphoreType.DMA,
            pltpu.SemaphoreType.DMA,
            pltpu.SemaphoreType.DMA,
            pltpu.SemaphoreType.DMA,
        ],
    )
    def sc_kernel(logits_hbm, tid_hbm, out_hbm,
                  tid_v, buf_0, buf_1, buf_2, buf_3, stage_v,
                  sem_0, sem_1, sem_2, sem_3):
        cid = lax.axis_index("c")
        sid = lax.axis_index("s")
        wid = sid * _NC + cid
        base = wid * per_w
        b_idx = base // seq

        pltpu.sync_copy(tid_hbm, tid_v)

        nbuf = 4
        bufs = (buf_0, buf_1, buf_2, buf_3)
        sems = (sem_0, sem_1, sem_2, sem_3)

        def fire(g):
            s0 = base % seq + g * _L
            vec = tid_v[b_idx, pl.ds(s0, _L)]
            buf, sem = bufs[g % nbuf], sems[g % nbuf]
            handles = []
            for k in range(_L):
                j = g * _L + k
                v = lax.reshape(lax.slice(vec, (k,), (k + 1,)), ())
                c0 = pl.multiple_of(jnp.bitwise_and(v, jnp.int32(-128)), 128)
                t0 = pl.multiple_of((base + j) & jnp.int32(-8), 8)
                handles.append(pltpu.async_copy(
                    logits_hbm.at[pl.ds(t0, 8), pl.ds(c0, 128)],
                    buf.at[pl.ds(k * 8, 8)],
                    sem,
                ))
            return vec, handles

        lane = lax.iota(jnp.int32, _L)
        rows = lane * 8 + (lane & 7)
        acc = jnp.zeros((_L,), jnp.float32)
        pending = [fire(g) for g in range(min(nbuf - 1, n_vec))]
        for g in range(n_vec):
            vec, handles = pending.pop(0)
            nxt = g + nbuf - 1
            if nxt < n_vec:
                pending.append(fire(nxt))
            for h in handles:
                h.wait()
            acc = acc + plsc.load_gather(bufs[g % nbuf], [rows, vec & 127])
        stage_v[...] = acc

        pltpu.sync_copy(stage_v, out_hbm.at[wid])

    return sc_kernel


def kernel(logits, target_ids):
    batch, seq, vocab = logits.shape
    logits2d = logits.reshape((batch * seq, vocab))
    tid = target_ids.astype(jnp.int32)
    partials = _make_sc_gather_sum(batch, seq, vocab)(logits2d, tid)
    return -(jnp.sum(partials) / batch)


# speedup vs baseline: 11.4107x; 1.0113x over previous
import functools

import jax
import jax.numpy as jnp
from jax import lax
from jax.experimental import pallas as pl
from jax.experimental.pallas import tpu as pltpu
from jax.experimental.pallas import tpu_sc as plsc

_INFO = plsc.get_sparse_core_info()
_NC, _NS, _L = _INFO.num_cores, _INFO.num_subcores, _INFO.num_lanes
_NW = _NC * _NS


@functools.lru_cache(maxsize=None)
def _make_sc_gather_sum(batch: int, seq: int, vocab: int):
    n_rows = batch * seq
    per_w = n_rows // _NW
    n_vec = per_w // _L
    mesh = plsc.VectorSubcoreMesh(core_axis_name="c", subcore_axis_name="s")

    @functools.partial(
        pl.kernel,
        mesh=mesh,
        compiler_params=pltpu.CompilerParams(needs_layout_passes=False),
        out_type=jax.ShapeDtypeStruct((_NW, _L), jnp.float32),
        scratch_types=[
            pltpu.VMEM((batch, seq), jnp.int32),
            pltpu.VMEM((_L * 8, 128), jnp.float32),
            pltpu.VMEM((_L * 8, 128), jnp.float32),
            pltpu.VMEM((_L * 8, 128), jnp.float32),
            pltpu.VMEM((_L * 8, 128), jnp.float32),
            pltpu.VMEM((_L * 8, 128), jnp.float32),
            pltpu.VMEM((_L * 8, 128), jnp.float32),
            pltpu.VMEM((_L,), jnp.float32),
            pltpu.SemaphoreType.DMA,
            pltpu.SemaphoreType.DMA,
            pltpu.SemaphoreType.DMA,
            pltpu.SemaphoreType.DMA,
            pltpu.SemaphoreType.DMA,
            pltpu.SemaphoreType.DMA,
        ],
    )
    def sc_kernel(logits_hbm, tid_hbm, out_hbm,
                  tid_v, buf_0, buf_1, buf_2, buf_3, buf_4, buf_5, stage_v,
                  sem_0, sem_1, sem_2, sem_3, sem_4, sem_5):
        cid = lax.axis_index("c")
        sid = lax.axis_index("s")
        wid = sid * _NC + cid
        base = wid * per_w
        b_idx = base // seq

        pltpu.sync_copy(tid_hbm, tid_v)

        nbuf = 6
        bufs = (buf_0, buf_1, buf_2, buf_3, buf_4, buf_5)
        sems = (sem_0, sem_1, sem_2, sem_3, sem_4, sem_5)

        def fire(g):
            s0 = base % seq + g * _L
            vec = tid_v[b_idx, pl.ds(s0, _L)]
            buf, sem = bufs[g % nbuf], sems[g % nbuf]
            handles = []
            for k in range(_L):
                j = g * _L + k
                v = lax.reshape(lax.slice(vec, (k,), (k + 1,)), ())
                c0 = pl.multiple_of(jnp.bitwise_and(v, jnp.int32(-128)), 128)
                t0 = pl.multiple_of((base + j) & jnp.int32(-8), 8)
                handles.append(pltpu.async_copy(
                    logits_hbm.at[pl.ds(t0, 8), pl.ds(c0, 128)],
                    buf.at[pl.ds(k * 8, 8)],
                    sem,
                ))
            return vec, handles

        lane = lax.iota(jnp.int32, _L)
        rows = lane * 8 + (lane & 7)
        acc = jnp.zeros((_L,), jnp.float32)
        pending = [fire(g) for g in range(min(nbuf - 1, n_vec))]
        for g in range(n_vec):
            vec, handles = pending.pop(0)
            nxt = g + nbuf - 1
            if nxt < n_vec:
                pending.append(fire(nxt))
            for h in handles:
                h.wait()
            acc = acc + plsc.load_gather(bufs[g % nbuf], [rows, vec & 127])
        stage_v[...] = acc

        pltpu.sync_copy(stage_v, out_hbm.at[wid])

    return sc_kernel


def kernel(logits, target_ids):
    batch, seq, vocab = logits.shape
    logits2d = logits.reshape((batch * seq, vocab))
    tid = target_ids.astype(jnp.int32)
    partials = _make_sc_gather_sum(batch, seq, vocab)(logits2d, tid)
    return -(jnp.sum(partials) / batch)
